# Initial kernel scaffold; baseline (speedup 1.0000x reference)
import jax
import jax.numpy as jnp
from jax.experimental import pallas as pl


def kernel(x, edge_index, W1l, b1, W1r, W2l, b2, W2r):
    raise NotImplementedError("write your pallas kernel here")


# speedup vs baseline: 7.7422x; 7.7422x over previous
import functools
import jax
import jax.numpy as jnp
from jax import lax
from jax.experimental import pallas as pl
from jax.experimental.pallas import tpu as pltpu
from jax.experimental.pallas import tpu_sc as plsc

N, E, DIN, H, C = 10000, 320000, 128, 256, 42
D1 = 144
D2 = 48
NTILES = 32
EPT = E // NTILES
CHUNK = 80
NCHUNK = EPT // CHUNK
NP = 10112
RPT = NP // 16


def _make_seg_sum(D):
    mesh = plsc.VectorSubcoreMesh(core_axis_name="c", subcore_axis_name="s")

    @functools.partial(
        pl.kernel,
        mesh=mesh,
        out_type=jax.ShapeDtypeStruct((2, NP, D), jnp.float32),
        compiler_params=pltpu.CompilerParams(use_tc_tiling_on_sc=False),
        scratch_types=[
            pltpu.VMEM((NCHUNK, CHUNK), jnp.int32),
            pltpu.VMEM((NCHUNK, CHUNK), jnp.int32),
            pltpu.VMEM((CHUNK, D), jnp.float32),
            pltpu.VMEM_SHARED((NP, D), jnp.float32),
            pltpu.SemaphoreType.DMA,
        ],
    )
    def seg(table_hbm, src_hbm, dst_hbm, zeros_hbm, out_hbm,
            srcv, dstv, rows, acc, sem):
        c = lax.axis_index("c")
        s = lax.axis_index("s")
        wid = c * 16 + s
        rbase = s * RPT
        pltpu.sync_copy(zeros_hbm, acc.at[pl.ds(rbase, RPT)])
        pltpu.sync_copy(src_hbm.at[wid], srcv)
        pltpu.sync_copy(dst_hbm.at[wid], dstv)
        plsc.subcore_barrier()

        def body(i, carry):
            pltpu.async_copy(table_hbm.at[srcv.at[i]], rows, sem).wait()
            pltpu.sync_copy(rows, acc.at[dstv.at[i]], add=True)
            return carry

        lax.fori_loop(0, NCHUNK, body, 0)
        plsc.subcore_barrier()
        pltpu.sync_copy(acc.at[pl.ds(rbase, RPT)],
                        out_hbm.at[c, pl.ds(rbase, RPT)])

    return seg


_seg1 = _make_seg_sum(D1)
_seg2 = _make_seg_sum(D2)

_BR = 1000


def _dense1_body(x_ref, p0_ref, p1_ref, w1l_ref, b1_ref, w1r_ref, w2l_ref,
                 h_ref, pp_ref):
    p = p0_ref[...] + p1_ref[...]
    cnt = jnp.maximum(p[:, DIN:DIN + 1], 1.0)
    mean = p[:, :DIN] / cnt
    z = (lax.dot(mean, w1l_ref[...], precision=lax.Precision.HIGHEST)
         + lax.dot(x_ref[...], w1r_ref[...], precision=lax.Precision.HIGHEST)
         + b1_ref[...])
    h = jnp.maximum(z, 0.0)
    h_ref[...] = h
    pp_ref[...] = lax.dot(h, w2l_ref[...], precision=lax.Precision.HIGHEST)


def _dense1(x, p0, p1, w1l_t, b1, w1r_t, w2l_t):
    grid = (N // _BR,)
    return pl.pallas_call(
        _dense1_body,
        grid=grid,
        in_specs=[
            pl.BlockSpec((_BR, DIN), lambda i: (i, 0)),
            pl.BlockSpec((_BR, D1), lambda i: (i, 0)),
            pl.BlockSpec((_BR, D1), lambda i: (i, 0)),
            pl.BlockSpec((DIN, H), lambda i: (0, 0)),
            pl.BlockSpec((1, H), lambda i: (0, 0)),
            pl.BlockSpec((DIN, H), lambda i: (0, 0)),
            pl.BlockSpec((H, D2), lambda i: (0, 0)),
        ],
        out_specs=[
            pl.BlockSpec((_BR, H), lambda i: (i, 0)),
            pl.BlockSpec((_BR, D2), lambda i: (i, 0)),
        ],
        out_shape=[
            jax.ShapeDtypeStruct((N, H), jnp.float32),
            jax.ShapeDtypeStruct((N, D2), jnp.float32),
        ],
    )(x, p0, p1, w1l_t, b1, w1r_t, w2l_t)


def _dense2_body(q0_ref, q1_ref, h_ref, c0_ref, c1_ref, w2r_ref, b2_ref,
                 out_ref):
    q = q0_ref[:, :C] + q1_ref[:, :C]
    cnt = jnp.maximum(c0_ref[:, 0:1] + c1_ref[:, 0:1], 1.0)
    z = (q / cnt + b2_ref[...]
         + lax.dot(h_ref[...], w2r_ref[...], precision=lax.Precision.HIGHEST))
    z = z - jnp.max(z, axis=1, keepdims=True)
    out_ref[...] = z - jnp.log(jnp.sum(jnp.exp(z), axis=1, keepdims=True))


def _dense2(q0, q1, h, c0, c1, w2r_t, b2):
    grid = (N // _BR,)
    return pl.pallas_call(
        _dense2_body,
        grid=grid,
        in_specs=[
            pl.BlockSpec((_BR, D2), lambda i: (i, 0)),
            pl.BlockSpec((_BR, D2), lambda i: (i, 0)),
            pl.BlockSpec((_BR, H), lambda i: (i, 0)),
            pl.BlockSpec((_BR, 8), lambda i: (i, 0)),
            pl.BlockSpec((_BR, 8), lambda i: (i, 0)),
            pl.BlockSpec((H, C), lambda i: (0, 0)),
            pl.BlockSpec((1, C), lambda i: (0, 0)),
        ],
        out_specs=pl.BlockSpec((_BR, C), lambda i: (i, 0)),
        out_shape=jax.ShapeDtypeStruct((N, C), jnp.float32),
    )(q0, q1, h, c0, c1, w2r_t, b2)


@jax.jit
def kernel(x, edge_index, W1l, b1, W1r, W2l, b2, W2r):
    src = edge_index[0].reshape(NTILES, NCHUNK, CHUNK)
    dst = edge_index[1].reshape(NTILES, NCHUNK, CHUNK)
    xa = jnp.concatenate(
        [x, jnp.ones((N, 1), jnp.float32), jnp.zeros((N, D1 - DIN - 1), jnp.float32)],
        axis=1)
    z1 = jnp.zeros((RPT, D1), jnp.float32)
    z2 = jnp.zeros((RPT, D2), jnp.float32)

    p = _seg1(xa, src, dst, z1)
    p0, p1 = p[0, :N], p[1, :N]
    h, pp = _dense1(x, p0, p1,
                    W1l.T, b1.reshape(1, H), W1r.T,
                    jnp.pad(W2l, ((0, D2 - C), (0, 0))).T)
    q = _seg2(pp, src, dst, z2)
    c0 = lax.slice(p0, (0, DIN), (N, DIN + 8))
    c1 = lax.slice(p1, (0, DIN), (N, DIN + 8))
    out = _dense2(q[0, :N], q[1, :N], h, c0, c1, W2r.T, b2.reshape(1, C))
    return out
